# baseline (speedup 1.0000x reference)
import functools

import jax
import jax.numpy as jnp
from jax.experimental import pallas as pl

B, S, H, D, K = 4, 2048, 1024, 64, 1024
R = 512


def _dist(r, cb_t, cb_sq):
    rr = jnp.sum(r * r, axis=-1, keepdims=True)
    cross = jax.lax.dot_general(
        r, cb_t, (((1,), (0,)), ((), ())), preferred_element_type=jnp.float32)
    return rr - 2.0 * cross + cb_sq


def _onehot_lookup(idx, cb):
    oh = (jax.lax.broadcasted_iota(jnp.int32, (idx.shape[0], K), 1)
          == idx[:, None]).astype(jnp.float32)
    return jax.lax.dot_general(
        oh, cb, (((1,), (0,)), ((), ())), preferred_element_type=jnp.float32,
        precision=jax.lax.Precision.HIGHEST)


def _rvq_kernel(x_ref, w_in_ref, b_in_ref, cb1_ref, cb2_ref, w_out_ref,
                b_out_ref, out_ref, q1_ref, q2_ref, l1_ref, l2_ref):
    h = jax.lax.dot_general(
        x_ref[...], w_in_ref[...], (((1,), (0,)), ((), ())),
        preferred_element_type=jnp.float32) + b_in_ref[...]

    cb1 = cb1_ref[...]
    cb1_sq = jnp.sum(cb1 * cb1, axis=-1)
    d1 = _dist(h, cb1.T, cb1_sq)
    idx1 = jnp.argmin(d1, axis=-1)
    q1 = _onehot_lookup(idx1, cb1)

    r2 = h - q1
    cb2 = cb2_ref[...]
    cb2_sq = jnp.sum(cb2 * cb2, axis=-1)
    d2 = _dist(r2, cb2.T, cb2_sq)
    idx2 = jnp.argmin(d2, axis=-1)
    q2 = _onehot_lookup(idx2, cb2)

    qsum = q1 + q2
    out_ref[...] = jax.lax.dot_general(
        qsum, w_out_ref[...], (((1,), (0,)), ((), ())),
        preferred_element_type=jnp.float32) + b_out_ref[...]
    q1_ref[...] = q1
    q2_ref[...] = q2

    e1 = q1 - h
    e2 = q2 - r2

    @pl.when(pl.program_id(0) == 0)
    def _init():
        l1_ref[...] = jnp.zeros_like(l1_ref)
        l2_ref[...] = jnp.zeros_like(l2_ref)

    l1_ref[...] += jnp.sum(e1 * e1).reshape(1, 1)
    l2_ref[...] += jnp.sum(e2 * e2).reshape(1, 1)


@jax.jit
def kernel(x, W_in, b_in, cb1, cb2, W_out, b_out):
    n = B * S
    x2 = x.reshape(n, H)
    grid = (n // R,)
    out, q1, q2, l1, l2 = pl.pallas_call(
        _rvq_kernel,
        grid=grid,
        in_specs=[
            pl.BlockSpec((R, H), lambda i: (i, 0)),
            pl.BlockSpec((H, D), lambda i: (0, 0)),
            pl.BlockSpec((1, D), lambda i: (0, 0)),
            pl.BlockSpec((K, D), lambda i: (0, 0)),
            pl.BlockSpec((K, D), lambda i: (0, 0)),
            pl.BlockSpec((D, H), lambda i: (0, 0)),
            pl.BlockSpec((1, H), lambda i: (0, 0)),
        ],
        out_specs=[
            pl.BlockSpec((R, H), lambda i: (i, 0)),
            pl.BlockSpec((R, D), lambda i: (i, 0)),
            pl.BlockSpec((R, D), lambda i: (i, 0)),
            pl.BlockSpec((1, 1), lambda i: (0, 0)),
            pl.BlockSpec((1, 1), lambda i: (0, 0)),
        ],
        out_shape=[
            jax.ShapeDtypeStruct((n, H), jnp.float32),
            jax.ShapeDtypeStruct((n, D), jnp.float32),
            jax.ShapeDtypeStruct((n, D), jnp.float32),
            jax.ShapeDtypeStruct((1, 1), jnp.float32),
            jax.ShapeDtypeStruct((1, 1), jnp.float32),
        ],
    )(x2, W_in, b_in.reshape(1, D), cb1, cb2, W_out, b_out.reshape(1, H))
    com = (l1[0, 0] + l2[0, 0]) / (2.0 * n * D)
    return (out.reshape(B, S, H), q1.reshape(B, S, D), q2.reshape(B, S, D),
            com)


# speedup vs baseline: 1.6648x; 1.6648x over previous
import functools

import jax
import jax.numpy as jnp
from jax.experimental import pallas as pl

B, S, H, D, K = 4, 2048, 1024, 64, 1024
R = 512


def _dist(r, cb_t, cb_sq):
    rr = jnp.sum(r * r, axis=-1, keepdims=True)
    cross = jax.lax.dot_general(
        r, cb_t, (((1,), (0,)), ((), ())), preferred_element_type=jnp.float32)
    return rr - 2.0 * cross + cb_sq


def _onehot_lookup(idx, cb_splits):
    oh = (jax.lax.broadcasted_iota(jnp.int32, (idx.shape[0], K), 1)
          == idx[:, None]).astype(jnp.bfloat16)
    y = jax.lax.dot_general(
        oh, cb_splits, (((1,), (0,)), ((), ())),
        preferred_element_type=jnp.float32)
    return (y[:, :D] + y[:, D:2 * D]) + y[:, 2 * D:]


def _rvq_kernel(x_ref, w_in_ref, b_in_ref, cb1_ref, cb2_ref, cbs1_ref,
                cbs2_ref, w_out_ref, b_out_ref, out_ref, q1_ref, q2_ref,
                l1_ref, l2_ref):
    h = jax.lax.dot_general(
        x_ref[...], w_in_ref[...], (((1,), (0,)), ((), ())),
        preferred_element_type=jnp.float32) + b_in_ref[...]

    cb1 = cb1_ref[...]
    cb1_sq = jnp.sum(cb1 * cb1, axis=-1)
    d1 = _dist(h, cb1.T, cb1_sq)
    idx1 = jnp.argmin(d1, axis=-1)
    q1 = _onehot_lookup(idx1, cbs1_ref[...])

    r2 = h - q1
    cb2 = cb2_ref[...]
    cb2_sq = jnp.sum(cb2 * cb2, axis=-1)
    d2 = _dist(r2, cb2.T, cb2_sq)
    idx2 = jnp.argmin(d2, axis=-1)
    q2 = _onehot_lookup(idx2, cbs2_ref[...])

    qsum = q1 + q2
    out_ref[...] = jax.lax.dot_general(
        qsum, w_out_ref[...], (((1,), (0,)), ((), ())),
        preferred_element_type=jnp.float32) + b_out_ref[...]
    q1_ref[...] = q1
    q2_ref[...] = q2

    e1 = q1 - h
    e2 = q2 - r2

    @pl.when(pl.program_id(0) == 0)
    def _init():
        l1_ref[...] = jnp.zeros_like(l1_ref)
        l2_ref[...] = jnp.zeros_like(l2_ref)

    l1_ref[...] += jnp.sum(e1 * e1).reshape(1, 1)
    l2_ref[...] += jnp.sum(e2 * e2).reshape(1, 1)


def _split3(cb):
    hi = cb.astype(jnp.bfloat16)
    r1 = cb - hi.astype(jnp.float32)
    mid = r1.astype(jnp.bfloat16)
    lo = (r1 - mid.astype(jnp.float32)).astype(jnp.bfloat16)
    return jnp.concatenate([hi, mid, lo], axis=1)


@jax.jit
def kernel(x, W_in, b_in, cb1, cb2, W_out, b_out):
    n = B * S
    x2 = x.reshape(n, H)
    cbs1 = _split3(cb1)
    cbs2 = _split3(cb2)
    grid = (n // R,)
    out, q1, q2, l1, l2 = pl.pallas_call(
        _rvq_kernel,
        grid=grid,
        in_specs=[
            pl.BlockSpec((R, H), lambda i: (i, 0)),
            pl.BlockSpec((H, D), lambda i: (0, 0)),
            pl.BlockSpec((1, D), lambda i: (0, 0)),
            pl.BlockSpec((K, D), lambda i: (0, 0)),
            pl.BlockSpec((K, D), lambda i: (0, 0)),
            pl.BlockSpec((K, 3 * D), lambda i: (0, 0)),
            pl.BlockSpec((K, 3 * D), lambda i: (0, 0)),
            pl.BlockSpec((D, H), lambda i: (0, 0)),
            pl.BlockSpec((1, H), lambda i: (0, 0)),
        ],
        out_specs=[
            pl.BlockSpec((R, H), lambda i: (i, 0)),
            pl.BlockSpec((R, D), lambda i: (i, 0)),
            pl.BlockSpec((R, D), lambda i: (i, 0)),
            pl.BlockSpec((1, 1), lambda i: (0, 0)),
            pl.BlockSpec((1, 1), lambda i: (0, 0)),
        ],
        out_shape=[
            jax.ShapeDtypeStruct((n, H), jnp.float32),
            jax.ShapeDtypeStruct((n, D), jnp.float32),
            jax.ShapeDtypeStruct((n, D), jnp.float32),
            jax.ShapeDtypeStruct((1, 1), jnp.float32),
            jax.ShapeDtypeStruct((1, 1), jnp.float32),
        ],
    )(x2, W_in, b_in.reshape(1, D), cb1, cb2, cbs1, cbs2, W_out,
      b_out.reshape(1, H))
    com = (l1[0, 0] + l2[0, 0]) / (2.0 * n * D)
    return (out.reshape(B, S, H), q1.reshape(B, S, D), q2.reshape(B, S, D),
            com)


# speedup vs baseline: 1.7814x; 1.0700x over previous
import functools

import jax
import jax.numpy as jnp
from jax.experimental import pallas as pl

B, S, H, D, K = 4, 2048, 1024, 64, 1024
R = 1024


def _dist(r, cb_t, cb_sq):
    rr = jnp.sum(r * r, axis=-1, keepdims=True)
    cross = jax.lax.dot_general(
        r, cb_t, (((1,), (0,)), ((), ())), preferred_element_type=jnp.float32)
    return rr - 2.0 * cross + cb_sq


def _onehot_lookup(idx, cb_splits):
    oh = (jax.lax.broadcasted_iota(jnp.int32, (idx.shape[0], K), 1)
          == idx[:, None]).astype(jnp.bfloat16)
    y = jax.lax.dot_general(
        oh, cb_splits, (((1,), (0,)), ((), ())),
        preferred_element_type=jnp.float32)
    return (y[:, :D] + y[:, D:2 * D]) + y[:, 2 * D:]


def _rvq_kernel(x_ref, w_in_ref, b_in_ref, cb1_ref, cb2_ref, cbs1_ref,
                cbs2_ref, w_out_ref, b_out_ref, out_ref, q1_ref, q2_ref,
                l1_ref, l2_ref):
    h = jax.lax.dot_general(
        x_ref[...], w_in_ref[...], (((1,), (0,)), ((), ())),
        preferred_element_type=jnp.float32) + b_in_ref[...]

    cb1 = cb1_ref[...]
    cb1_sq = jnp.sum(cb1 * cb1, axis=-1)
    d1 = _dist(h, cb1.T, cb1_sq)
    idx1 = jnp.argmin(d1, axis=-1)
    q1 = _onehot_lookup(idx1, cbs1_ref[...])

    r2 = h - q1
    cb2 = cb2_ref[...]
    cb2_sq = jnp.sum(cb2 * cb2, axis=-1)
    d2 = _dist(r2, cb2.T, cb2_sq)
    idx2 = jnp.argmin(d2, axis=-1)
    q2 = _onehot_lookup(idx2, cbs2_ref[...])

    qsum = q1 + q2
    out_ref[...] = jax.lax.dot_general(
        qsum, w_out_ref[...], (((1,), (0,)), ((), ())),
        preferred_element_type=jnp.float32) + b_out_ref[...]
    q1_ref[...] = q1
    q2_ref[...] = q2

    e1 = q1 - h
    e2 = q2 - r2

    @pl.when(pl.program_id(0) == 0)
    def _init():
        l1_ref[...] = jnp.zeros_like(l1_ref)
        l2_ref[...] = jnp.zeros_like(l2_ref)

    l1_ref[...] += jnp.sum(e1 * e1).reshape(1, 1)
    l2_ref[...] += jnp.sum(e2 * e2).reshape(1, 1)


def _split3(cb):
    hi = cb.astype(jnp.bfloat16)
    r1 = cb - hi.astype(jnp.float32)
    mid = r1.astype(jnp.bfloat16)
    lo = (r1 - mid.astype(jnp.float32)).astype(jnp.bfloat16)
    return jnp.concatenate([hi, mid, lo], axis=1)


@jax.jit
def kernel(x, W_in, b_in, cb1, cb2, W_out, b_out):
    n = B * S
    x2 = x.reshape(n, H)
    cbs1 = _split3(cb1)
    cbs2 = _split3(cb2)
    grid = (n // R,)
    out, q1, q2, l1, l2 = pl.pallas_call(
        _rvq_kernel,
        grid=grid,
        in_specs=[
            pl.BlockSpec((R, H), lambda i: (i, 0)),
            pl.BlockSpec((H, D), lambda i: (0, 0)),
            pl.BlockSpec((1, D), lambda i: (0, 0)),
            pl.BlockSpec((K, D), lambda i: (0, 0)),
            pl.BlockSpec((K, D), lambda i: (0, 0)),
            pl.BlockSpec((K, 3 * D), lambda i: (0, 0)),
            pl.BlockSpec((K, 3 * D), lambda i: (0, 0)),
            pl.BlockSpec((D, H), lambda i: (0, 0)),
            pl.BlockSpec((1, H), lambda i: (0, 0)),
        ],
        out_specs=[
            pl.BlockSpec((R, H), lambda i: (i, 0)),
            pl.BlockSpec((R, D), lambda i: (i, 0)),
            pl.BlockSpec((R, D), lambda i: (i, 0)),
            pl.BlockSpec((1, 1), lambda i: (0, 0)),
            pl.BlockSpec((1, 1), lambda i: (0, 0)),
        ],
        out_shape=[
            jax.ShapeDtypeStruct((n, H), jnp.float32),
            jax.ShapeDtypeStruct((n, D), jnp.float32),
            jax.ShapeDtypeStruct((n, D), jnp.float32),
            jax.ShapeDtypeStruct((1, 1), jnp.float32),
            jax.ShapeDtypeStruct((1, 1), jnp.float32),
        ],
    )(x2, W_in, b_in.reshape(1, D), cb1, cb2, cbs1, cbs2, W_out,
      b_out.reshape(1, H))
    com = (l1[0, 0] + l2[0, 0]) / (2.0 * n * D)
    return (out.reshape(B, S, H), q1.reshape(B, S, D), q2.reshape(B, S, D),
            com)
